# baseline (speedup 1.0000x reference)
import functools

import jax
import jax.numpy as jnp
from jax.experimental import pallas as pl
from jax.experimental.pallas import tpu as pltpu

NUM_GATES = 16
DIM = 4096
EPS = 1e-9
SECOND_THRESHOLD = 0.2
CAPACITY = 160
BN = 256


def _gating_kernel(x_ref, w_ref, p_ref, disp_ref, comb_ref, bal_ref, z_ref,
                   carry_ref, proxy_ref, accb_ref, accz_ref, *, nb_total):
    b = pl.program_id(0)
    nb = pl.program_id(1)

    @pl.when(nb == 0)
    def _reset_batch():
        carry_ref[...] = jnp.zeros_like(carry_ref)
        proxy_ref[...] = jnp.zeros_like(proxy_ref)

    @pl.when((b == 0) & (nb == 0))
    def _reset_all():
        accb_ref[...] = jnp.zeros_like(accb_ref)
        accz_ref[...] = jnp.zeros_like(accz_ref)

    xb = x_ref[0]
    logits = jax.lax.dot_general(
        xb, w_ref[...], (((1,), (0,)), ((), ())),
        preferred_element_type=jnp.float32)

    m = jnp.max(logits, axis=1, keepdims=True)
    ex = jnp.exp(logits - m)
    s = jnp.sum(ex, axis=1, keepdims=True)
    sm = ex / s
    lse = m + jnp.log(s)

    accz_ref[...] = accz_ref[...] + jnp.sum(lse, axis=(0, 1), keepdims=True)
    proxy_ref[...] = proxy_ref[...] + jnp.sum(sm, axis=0, keepdims=True)

    e_iota = jax.lax.broadcasted_iota(jnp.int32, (BN, NUM_GATES), 1)
    g1 = jnp.max(sm, axis=1, keepdims=True)
    i1 = jnp.min(jnp.where(sm == g1, e_iota, NUM_GATES), axis=1, keepdims=True)
    sm2 = jnp.where(e_iota == i1, -jnp.inf, sm)
    g2 = jnp.max(sm2, axis=1, keepdims=True)
    i2 = jnp.min(jnp.where(sm2 == g2, e_iota, NUM_GATES), axis=1, keepdims=True)

    denom = g1 + g2 + EPS
    g1n = g1 / denom
    g2n = g2 / denom

    probs = p_ref[0, 0]
    keep2 = probs < (g2n / jnp.float32(SECOND_THRESHOLD))

    mask1 = (e_iota == i1).astype(jnp.float32)
    mask2 = ((e_iota == i2) & keep2).astype(jnp.float32)

    tri = (jax.lax.broadcasted_iota(jnp.int32, (BN, BN), 0)
           > jax.lax.broadcasted_iota(jnp.int32, (BN, BN), 1)).astype(jnp.float32)
    excl1 = jax.lax.dot_general(
        tri, mask1, (((1,), (0,)), ((), ())), preferred_element_type=jnp.float32)
    excl2 = jax.lax.dot_general(
        tri, mask2, (((1,), (0,)), ((), ())), preferred_element_type=jnp.float32)

    carry1 = carry_ref[0:1, :]
    carry2 = carry_ref[1:2, :]
    pos1 = jnp.sum((excl1 + carry1) * mask1, axis=1, keepdims=True)
    pos2 = jnp.sum((excl2 + carry2) * mask2, axis=1, keepdims=True)
    carry_ref[0:1, :] = carry1 + jnp.sum(mask1, axis=0, keepdims=True)
    carry_ref[1:2, :] = carry2 + jnp.sum(mask2, axis=0, keepdims=True)

    kept1 = (pos1 < CAPACITY).astype(jnp.float32)
    kept2 = (keep2 & (pos2 < CAPACITY)).astype(jnp.float32)
    g1f = g1n * kept1
    g2f = g2n * kept2

    idx1 = i1 * CAPACITY + pos1.astype(jnp.int32)
    idx2 = i2 * CAPACITY + pos2.astype(jnp.int32)

    c_iota = jax.lax.broadcasted_iota(jnp.int32, (BN, NUM_GATES * CAPACITY), 1)
    combine = (jnp.where(c_iota == idx1, g1f, 0.0)
               + jnp.where(c_iota == idx2, g2f, 0.0))
    comb_ref[0] = combine
    disp_ref[0] = (combine != 0.0).astype(jnp.float32)

    @pl.when(nb == nb_total - 1)
    def _finish_batch():
        accb_ref[...] = accb_ref[...] + jnp.sum(
            proxy_ref[...] * carry_ref[0:1, :], axis=(0, 1), keepdims=True)

    bal_ref[...] = accb_ref[...] * jnp.float32(4.0 / (2048.0 * 2048.0))
    z_ref[...] = accz_ref[...] * jnp.float32(0.25)


@jax.jit
def kernel(x, w_gating):
    b, n, d = x.shape
    nb_total = n // BN
    probs = jax.lax.stop_gradient(
        jax.random.uniform(jax.random.key(42), (b, n), dtype=jnp.float32))
    probs4 = probs.reshape(b, nb_total, BN, 1)

    grid = (b, nb_total)
    flat = NUM_GATES * CAPACITY
    out_shape = [
        jax.ShapeDtypeStruct((b, n, flat), jnp.float32),
        jax.ShapeDtypeStruct((b, n, flat), jnp.float32),
        jax.ShapeDtypeStruct((1, 1), jnp.float32),
        jax.ShapeDtypeStruct((1, 1), jnp.float32),
    ]
    disp, comb, bal, z = pl.pallas_call(
        functools.partial(_gating_kernel, nb_total=nb_total),
        grid=grid,
        in_specs=[
            pl.BlockSpec((1, BN, d), lambda i, j: (i, j, 0)),
            pl.BlockSpec((d, NUM_GATES), lambda i, j: (0, 0)),
            pl.BlockSpec((1, 1, BN, 1), lambda i, j: (i, j, 0, 0)),
        ],
        out_specs=[
            pl.BlockSpec((1, BN, flat), lambda i, j: (i, j, 0)),
            pl.BlockSpec((1, BN, flat), lambda i, j: (i, j, 0)),
            pl.BlockSpec((1, 1), lambda i, j: (0, 0)),
            pl.BlockSpec((1, 1), lambda i, j: (0, 0)),
        ],
        out_shape=out_shape,
        scratch_shapes=[
            pltpu.VMEM((2, NUM_GATES), jnp.float32),
            pltpu.VMEM((1, NUM_GATES), jnp.float32),
            pltpu.VMEM((1, 1), jnp.float32),
            pltpu.VMEM((1, 1), jnp.float32),
        ],
    )(x, w_gating, probs4)

    dispatch = disp.reshape(b, n, NUM_GATES, CAPACITY)
    combine = comb.reshape(b, n, NUM_GATES, CAPACITY)
    return dispatch, combine, bal[0, 0], z[0, 0]


# speedup vs baseline: 1.0333x; 1.0333x over previous
import functools

import jax
import jax.numpy as jnp
from jax.experimental import pallas as pl
from jax.experimental.pallas import tpu as pltpu

NUM_GATES = 16
DIM = 4096
EPS = 1e-9
SECOND_THRESHOLD = 0.2
CAPACITY = 160
BN = 512


def _gating_kernel(x_ref, w_ref, p_ref, disp_ref, comb_ref, bal_ref, z_ref,
                   carry_ref, proxy_ref, accb_ref, accz_ref, *, nb_total):
    b = pl.program_id(0)
    nb = pl.program_id(1)

    @pl.when(nb == 0)
    def _reset_batch():
        carry_ref[...] = jnp.zeros_like(carry_ref)
        proxy_ref[...] = jnp.zeros_like(proxy_ref)

    @pl.when((b == 0) & (nb == 0))
    def _reset_all():
        accb_ref[...] = jnp.zeros_like(accb_ref)
        accz_ref[...] = jnp.zeros_like(accz_ref)

    xb = x_ref[0]
    logits = jax.lax.dot_general(
        xb, w_ref[...], (((1,), (0,)), ((), ())),
        preferred_element_type=jnp.float32)

    m = jnp.max(logits, axis=1, keepdims=True)
    ex = jnp.exp(logits - m)
    s = jnp.sum(ex, axis=1, keepdims=True)
    sm = ex / s
    lse = m + jnp.log(s)

    accz_ref[...] = accz_ref[...] + jnp.sum(lse, axis=(0, 1), keepdims=True)
    proxy_ref[...] = proxy_ref[...] + jnp.sum(sm, axis=0, keepdims=True)

    e_iota = jax.lax.broadcasted_iota(jnp.int32, (BN, NUM_GATES), 1)
    g1 = jnp.max(sm, axis=1, keepdims=True)
    i1 = jnp.min(jnp.where(sm == g1, e_iota, NUM_GATES), axis=1, keepdims=True)
    sm2 = jnp.where(e_iota == i1, -jnp.inf, sm)
    g2 = jnp.max(sm2, axis=1, keepdims=True)
    i2 = jnp.min(jnp.where(sm2 == g2, e_iota, NUM_GATES), axis=1, keepdims=True)

    denom = g1 + g2 + EPS
    g1n = g1 / denom
    g2n = g2 / denom

    probs = p_ref[0, 0]
    keep2 = probs < (g2n / jnp.float32(SECOND_THRESHOLD))

    mask1 = (e_iota == i1).astype(jnp.float32)
    mask2 = ((e_iota == i2) & keep2).astype(jnp.float32)

    tri = (jax.lax.broadcasted_iota(jnp.int32, (BN, BN), 0)
           > jax.lax.broadcasted_iota(jnp.int32, (BN, BN), 1)).astype(jnp.float32)
    excl1 = jax.lax.dot_general(
        tri, mask1, (((1,), (0,)), ((), ())), preferred_element_type=jnp.float32)
    excl2 = jax.lax.dot_general(
        tri, mask2, (((1,), (0,)), ((), ())), preferred_element_type=jnp.float32)

    carry1 = carry_ref[0:1, :]
    carry2 = carry_ref[1:2, :]
    pos1 = jnp.sum((excl1 + carry1) * mask1, axis=1, keepdims=True)
    pos2 = jnp.sum((excl2 + carry2) * mask2, axis=1, keepdims=True)
    carry_ref[0:1, :] = carry1 + jnp.sum(mask1, axis=0, keepdims=True)
    carry_ref[1:2, :] = carry2 + jnp.sum(mask2, axis=0, keepdims=True)

    kept1 = (pos1 < CAPACITY).astype(jnp.float32)
    kept2 = (keep2 & (pos2 < CAPACITY)).astype(jnp.float32)
    g1f = g1n * kept1
    g2f = g2n * kept2

    idx1 = i1 * CAPACITY + pos1.astype(jnp.int32)
    idx2 = i2 * CAPACITY + pos2.astype(jnp.int32)

    c_iota = jax.lax.broadcasted_iota(jnp.int32, (BN, NUM_GATES * CAPACITY), 1)
    combine = (jnp.where(c_iota == idx1, g1f, 0.0)
               + jnp.where(c_iota == idx2, g2f, 0.0))
    comb_ref[0] = combine
    disp_ref[0] = (combine != 0.0).astype(jnp.float32)

    @pl.when(nb == nb_total - 1)
    def _finish_batch():
        accb_ref[...] = accb_ref[...] + jnp.sum(
            proxy_ref[...] * carry_ref[0:1, :], axis=(0, 1), keepdims=True)

    bal_ref[...] = accb_ref[...] * jnp.float32(4.0 / (2048.0 * 2048.0))
    z_ref[...] = accz_ref[...] * jnp.float32(0.25)


@jax.jit
def kernel(x, w_gating):
    b, n, d = x.shape
    nb_total = n // BN
    probs = jax.lax.stop_gradient(
        jax.random.uniform(jax.random.key(42), (b, n), dtype=jnp.float32))
    probs4 = probs.reshape(b, nb_total, BN, 1)

    grid = (b, nb_total)
    flat = NUM_GATES * CAPACITY
    out_shape = [
        jax.ShapeDtypeStruct((b, n, flat), jnp.float32),
        jax.ShapeDtypeStruct((b, n, flat), jnp.float32),
        jax.ShapeDtypeStruct((1, 1), jnp.float32),
        jax.ShapeDtypeStruct((1, 1), jnp.float32),
    ]
    disp, comb, bal, z = pl.pallas_call(
        functools.partial(_gating_kernel, nb_total=nb_total),
        grid=grid,
        in_specs=[
            pl.BlockSpec((1, BN, d), lambda i, j: (i, j, 0)),
            pl.BlockSpec((d, NUM_GATES), lambda i, j: (0, 0)),
            pl.BlockSpec((1, 1, BN, 1), lambda i, j: (i, j, 0, 0)),
        ],
        out_specs=[
            pl.BlockSpec((1, BN, flat), lambda i, j: (i, j, 0)),
            pl.BlockSpec((1, BN, flat), lambda i, j: (i, j, 0)),
            pl.BlockSpec((1, 1), lambda i, j: (0, 0)),
            pl.BlockSpec((1, 1), lambda i, j: (0, 0)),
        ],
        out_shape=out_shape,
        scratch_shapes=[
            pltpu.VMEM((2, NUM_GATES), jnp.float32),
            pltpu.VMEM((1, NUM_GATES), jnp.float32),
            pltpu.VMEM((1, 1), jnp.float32),
            pltpu.VMEM((1, 1), jnp.float32),
        ],
    )(x, w_gating, probs4)

    dispatch = disp.reshape(b, n, NUM_GATES, CAPACITY)
    combine = comb.reshape(b, n, NUM_GATES, CAPACITY)
    return dispatch, combine, bal[0, 0], z[0, 0]
